# baseline (speedup 1.0000x reference)
import jax
import jax.numpy as jnp
from jax.experimental import pallas as pl

K = 64
ROW_BLOCK = 32
CHAINS = 1


def _topk_mask_kernel(x_ref, o_ref):
    x = x_ref[...]
    R = x.shape[0]
    b = jax.lax.bitcast_convert_type(x, jnp.uint32)
    key = jnp.where(b >= jnp.uint32(0x80000000), ~b, b | jnp.uint32(0x80000000))
    rs = R // CHAINS
    keys = [key[c * rs:(c + 1) * rs] for c in range(CHAINS)]
    thrs = [jnp.zeros((rs, 1), jnp.uint32) for _ in range(CHAINS)]
    for bit in range(31, -1, -1):
        for c in range(CHAINS):
            cand = thrs[c] | jnp.uint32(1 << bit)
            cnt = jnp.sum((keys[c] >= cand).astype(jnp.int32), axis=1,
                          keepdims=True)
            thrs[c] = jnp.where(cnt >= K, cand, thrs[c])
    thr = jnp.concatenate(thrs, axis=0)
    o_ref[...] = jnp.where(key >= thr, x, jnp.float32(0.0))


def kernel(x):
    B, N = x.shape
    return pl.pallas_call(
        _topk_mask_kernel,
        grid=(B // ROW_BLOCK,),
        in_specs=[pl.BlockSpec((ROW_BLOCK, N), lambda i: (i, 0))],
        out_specs=pl.BlockSpec((ROW_BLOCK, N), lambda i: (i, 0)),
        out_shape=jax.ShapeDtypeStruct((B, N), x.dtype),
    )(x)


# speedup vs baseline: 1.1389x; 1.1389x over previous
import jax
import jax.numpy as jnp
from jax.experimental import pallas as pl

K = 64
ROW_BLOCK = 64


def _topk_mask_kernel(x_ref, o_ref):
    x = x_ref[...]
    b = jax.lax.bitcast_convert_type(x, jnp.uint32)
    key = jnp.where(b >= jnp.uint32(0x80000000), ~b, b | jnp.uint32(0x80000000))
    thr = jnp.zeros((x.shape[0], 1), jnp.uint32)
    for bit in range(31, -1, -1):
        cand = thr | jnp.uint32(1 << bit)
        cnt = jnp.sum((key >= cand).astype(jnp.int32), axis=1, keepdims=True)
        thr = jnp.where(cnt >= K, cand, thr)
    o_ref[...] = jnp.where(key >= thr, x, jnp.float32(0.0))


def kernel(x):
    B, N = x.shape
    return pl.pallas_call(
        _topk_mask_kernel,
        grid=(B // ROW_BLOCK,),
        in_specs=[pl.BlockSpec((ROW_BLOCK, N), lambda i: (i, 0))],
        out_specs=pl.BlockSpec((ROW_BLOCK, N), lambda i: (i, 0)),
        out_shape=jax.ShapeDtypeStruct((B, N), x.dtype),
    )(x)


# speedup vs baseline: 1.1662x; 1.0239x over previous
import jax
import jax.numpy as jnp
from jax.experimental import pallas as pl

K = 64
ROW_BLOCK = 64


def _topk_mask_kernel(x_ref, o_ref):
    x = x_ref[...]
    b = jax.lax.bitcast_convert_type(x, jnp.uint32)
    key = jnp.where(b >= jnp.uint32(0x80000000), ~b, b | jnp.uint32(0x80000000))
    thr = jnp.zeros((x.shape[0], 1), jnp.uint32)
    for bit in range(31, -1, -1):
        cand = thr | jnp.uint32(1 << bit)
        cnt = jnp.sum((key >= cand).astype(jnp.float32), axis=1, keepdims=True)
        thr = jnp.where(cnt >= jnp.float32(K), cand, thr)
    o_ref[...] = jnp.where(key >= thr, x, jnp.float32(0.0))


def kernel(x):
    B, N = x.shape
    return pl.pallas_call(
        _topk_mask_kernel,
        grid=(B // ROW_BLOCK,),
        in_specs=[pl.BlockSpec((ROW_BLOCK, N), lambda i: (i, 0))],
        out_specs=pl.BlockSpec((ROW_BLOCK, N), lambda i: (i, 0)),
        out_shape=jax.ShapeDtypeStruct((B, N), x.dtype),
    )(x)
